# baseline (speedup 1.0000x reference)
import functools

import jax
import jax.numpy as jnp
import numpy as np
from jax import lax
from jax.experimental import pallas as pl
from jax.experimental.pallas import tpu as pltpu
from jax.experimental.pallas import tpu_sc as plsc

_F = 26
_E = 4
_V = 100000
_D = 16
_B = 4096

_NW = 32
_PAIRS = _F * _E
_BC = 512
_NBC = _B // _BC
_UNITS = _PAIRS * _NBC
_UPW = _UNITS // _NW
_G = 64
_NG = _BC // _G
_L = 16


def _sc_body(tab_hbm, x_hbm, out_hbm, xv, tv, sv, gbuf, wbuf, gsem, wsem):
    wid = lax.axis_index("s") * 2 + lax.axis_index("c")
    lanes = lax.iota(jnp.int32, _L)

    def unit(i, _):
        u = wid * _UPW + i
        p = u // _NBC
        f = p // _E
        m = p - f * _E
        b0 = (u - p * _NBC) * _BC
        tbase = p * (_V // 8)

        pltpu.sync_copy(x_hbm.at[f, m, pl.ds(b0, _BC)], xv)

        def prep(c, _):
            s = pl.ds(c * _L, _L)
            v = xv[s]
            tv[s] = tbase + lax.shift_right_logical(v, 2 + 1)
            sv[s] = lax.bitwise_and(v, jnp.int32(7))
            return 0

        lax.fori_loop(0, _BC // _L, prep, 0)

        def sub(g, _):
            g0 = g * _G
            pltpu.async_copy(
                tab_hbm.at[tv.at[pl.ds(g0, _G)]], gbuf, gsem
            ).wait()
            def ext(c, _):
                l0 = g0 + c * _L
                i0 = lanes + c * _L
                i1 = sv[pl.ds(l0, _L)] * _D
                for d in range(_D):
                    vals = plsc.load_gather(
                        gbuf, [i0, i1 + jnp.int32(d)]
                    )
                    plsc.store_scatter(
                        wbuf, [jnp.full((_L,), d, jnp.int32), lanes + l0], vals
                    )
                return 0

            lax.fori_loop(0, _G // _L, ext, 0)
            return 0

        lax.fori_loop(0, _NG, sub, 0)

        pltpu.async_copy(
            wbuf, out_hbm.at[m, pl.ds(f * _D, _D), pl.ds(b0, _BC)], wsem
        ).wait()
        return 0

    lax.fori_loop(0, _UPW, unit, 0)


@jax.jit
def kernel(x, tables):
    tab3 = tables.reshape(_F * _E * _V // 8, 8 * _D)
    xt = x.transpose(2, 1, 0)
    mesh = plsc.VectorSubcoreMesh(core_axis_name="c", subcore_axis_name="s")
    run = pl.kernel(
        _sc_body,
        mesh=mesh,
        out_type=jax.ShapeDtypeStruct((_E, _F * _D, _B), jnp.float32),
        scratch_types=[
            pltpu.VMEM((_BC,), jnp.int32),
            pltpu.VMEM((_BC,), jnp.int32),
            pltpu.VMEM((_BC,), jnp.int32),
            pltpu.VMEM((_G, 8 * _D), jnp.float32),
            pltpu.VMEM((_D, _BC), jnp.float32),
            pltpu.SemaphoreType.DMA,
            pltpu.SemaphoreType.DMA,
        ],
        compiler_params=pltpu.CompilerParams(
            use_tc_tiling_on_sc=True, needs_layout_passes=False
        ),
    )
    out = run(tab3, xt)
    return out.transpose(2, 0, 1)


# speedup vs baseline: 3.3821x; 3.3821x over previous
import functools

import jax
import jax.numpy as jnp
import numpy as np
from jax import lax
from jax.experimental import pallas as pl
from jax.experimental.pallas import tpu as pltpu
from jax.experimental.pallas import tpu_sc as plsc

_F = 26
_E = 4
_V = 100000
_D = 16
_B = 4096

_ROWS = _B * _E * _F
_NW = 32
_RPW = _ROWS // _NW
_CHUNK = 1664
_NCHUNK = _RPW // _CHUNK
_CW = _CHUNK * _D
_L = 16

_pat = ((np.arange(_F)[None, :] * _E + np.arange(_E)[:, None]) * (_D * _V))
_OFFSETS = np.tile(_pat.reshape(-1), _RPW // (_E * _F)).astype(np.int32)


def _sc_body(tab_hbm, x_hbm, off_hbm, out_hbm, xv, ov, idx_v, gbuf, sem):
    wid = lax.axis_index("s") * 2 + lax.axis_index("c")
    base = wid * _RPW
    pltpu.sync_copy(x_hbm.at[pl.ds(base, _RPW)], xv)
    pltpu.sync_copy(off_hbm, ov)
    lanes16 = lax.iota(jnp.int32, _L) * _D

    def chunk(c, _):
        c0 = c * _CHUNK

        def build(g, _):
            s = pl.ds(c0 + g * _L, _L)
            b16 = xv[s] + ov[s]
            k0 = g * (_L * _D)
            for d in range(_D):
                plsc.store_scatter(
                    idx_v,
                    [lanes16 + jnp.int32(k0 + d)],
                    b16 + jnp.int32(d * _V),
                )
            return 0

        lax.fori_loop(0, _CHUNK // _L, build, 0)
        pltpu.async_copy(tab_hbm.at[idx_v], gbuf, sem).wait()
        pltpu.sync_copy(gbuf, out_hbm.at[pl.ds((base + c0) * _D, _CW)])
        return 0

    lax.fori_loop(0, _NCHUNK, chunk, 0)


@jax.jit
def kernel(x, tables):
    tab1d = tables.transpose(0, 1, 3, 2).reshape(_F * _E * _D * _V)
    x_flat = x.reshape(_ROWS)
    mesh = plsc.VectorSubcoreMesh(core_axis_name="c", subcore_axis_name="s")
    run = pl.kernel(
        _sc_body,
        mesh=mesh,
        out_type=jax.ShapeDtypeStruct((_ROWS * _D,), jnp.float32),
        scratch_types=[
            pltpu.VMEM((_RPW,), jnp.int32),
            pltpu.VMEM((_RPW,), jnp.int32),
            pltpu.VMEM((_CW,), jnp.int32),
            pltpu.VMEM((_CW,), jnp.float32),
            pltpu.SemaphoreType.DMA,
        ],
        compiler_params=pltpu.CompilerParams(
            use_tc_tiling_on_sc=False, needs_layout_passes=False
        ),
    )
    out = run(tab1d, x_flat, jnp.asarray(_OFFSETS))
    return out.reshape(_B, _E, _F * _D)


# speedup vs baseline: 3.4018x; 1.0058x over previous
import functools

import jax
import jax.numpy as jnp
import numpy as np
from jax import lax
from jax.experimental import pallas as pl
from jax.experimental.pallas import tpu as pltpu
from jax.experimental.pallas import tpu_sc as plsc

_F = 26
_E = 4
_V = 100000
_D = 16
_B = 4096

_ROWS = _B * _E * _F
_NW = 32
_RPW = _ROWS // _NW
_CHUNK = 1024
_NCHUNK = _RPW // _CHUNK
_CW = _CHUNK * _D
_L = 16

_pat = ((np.arange(_F)[None, :] * _E + np.arange(_E)[:, None]) * (_D * _V))
_OFFSETS = np.tile(_pat.reshape(-1), _RPW // (_E * _F)).astype(np.int32)


def _sc_body(
    tab_hbm, x_hbm, off_hbm, out_hbm,
    xv, ov, idx_a, idx_b, gbuf_a, gbuf_b, sem_a, sem_b,
):
    wid = lax.axis_index("s") * 2 + lax.axis_index("c")
    base = wid * _RPW
    pltpu.sync_copy(x_hbm.at[pl.ds(base, _RPW)], xv)
    pltpu.sync_copy(off_hbm, ov)
    lanes16 = lax.iota(jnp.int32, _L) * _D
    idx_bufs = (idx_a, idx_b)
    gbufs = (gbuf_a, gbuf_b)
    sems = (sem_a, sem_b)

    def build(c, idx_v):
        c0 = c * _CHUNK

        def step(g, _):
            s = pl.ds(c0 + g * _L, _L)
            b16 = xv[s] + ov[s]
            k0 = g * (_L * _D)
            for d in range(_D):
                plsc.store_scatter(
                    idx_v,
                    [lanes16 + jnp.int32(k0 + d)],
                    b16 + jnp.int32(d * _V),
                )
            return 0

        lax.fori_loop(0, _CHUNK // _L, step, 0)

    build(0, idx_a)
    pltpu.async_copy(tab_hbm.at[idx_a], gbuf_a, sem_a)
    for c in range(_NCHUNK):
        cur = c % 2
        nxt = (c + 1) % 2
        if c + 1 < _NCHUNK:
            build(c + 1, idx_bufs[nxt])
        pltpu.make_async_copy(
            tab_hbm.at[idx_bufs[cur]], gbufs[cur], sems[cur]
        ).wait()
        if c + 1 < _NCHUNK:
            pltpu.async_copy(
                tab_hbm.at[idx_bufs[nxt]], gbufs[nxt], sems[nxt]
            )
        pltpu.sync_copy(
            gbufs[cur], out_hbm.at[pl.ds((base + c * _CHUNK) * _D, _CW)]
        )


@jax.jit
def kernel(x, tables):
    tab1d = tables.transpose(0, 1, 3, 2).reshape(_F * _E * _D * _V)
    x_flat = x.reshape(_ROWS)
    mesh = plsc.VectorSubcoreMesh(core_axis_name="c", subcore_axis_name="s")
    run = pl.kernel(
        _sc_body,
        mesh=mesh,
        out_type=jax.ShapeDtypeStruct((_ROWS * _D,), jnp.float32),
        scratch_types=[
            pltpu.VMEM((_RPW,), jnp.int32),
            pltpu.VMEM((_RPW,), jnp.int32),
            pltpu.VMEM((_CW,), jnp.int32),
            pltpu.VMEM((_CW,), jnp.int32),
            pltpu.VMEM((_CW,), jnp.float32),
            pltpu.VMEM((_CW,), jnp.float32),
            pltpu.SemaphoreType.DMA,
            pltpu.SemaphoreType.DMA,
        ],
        compiler_params=pltpu.CompilerParams(
            use_tc_tiling_on_sc=False, needs_layout_passes=False
        ),
    )
    out = run(tab1d, x_flat, jnp.asarray(_OFFSETS))
    return out.reshape(_B, _E, _F * _D)


# speedup vs baseline: 3.4051x; 1.0010x over previous
import jax
import jax.numpy as jnp
import numpy as np
from jax import lax
from jax.experimental import pallas as pl
from jax.experimental.pallas import tpu as pltpu
from jax.experimental.pallas import tpu_sc as plsc

_F = 26
_E = 4
_V = 100000
_D = 16
_B = 4096

_ROWS = _B * _E * _F
_NW = 32
_RPW = _ROWS // _NW
_CHUNK = 1024
_NCHUNK = _RPW // _CHUNK
_CW = _CHUNK * _D
_L = 16

_pat = ((np.arange(_F)[None, :] * _E + np.arange(_E)[:, None]) * (_D * _V))
_OFFSETS = np.tile(_pat.reshape(-1), _RPW // (_E * _F)).astype(np.int32)


def _sc_body(
    tab_hbm, x_hbm, off_hbm, out_hbm,
    xv, ov, idx_a, idx_b, gbuf_a, gbuf_b, sem_a, sem_b,
):
    wid = lax.axis_index("s") * 2 + lax.axis_index("c")
    base = wid * _RPW
    pltpu.sync_copy(x_hbm.at[pl.ds(base, _RPW)], xv)
    pltpu.sync_copy(off_hbm, ov)
    lanes16 = lax.iota(jnp.int32, _L) * _D
    idx_bufs = (idx_a, idx_b)
    gbufs = (gbuf_a, gbuf_b)
    sems = (sem_a, sem_b)

    def build(c, idx_v):
        c0 = c * _CHUNK

        def step(g, _):
            s = pl.ds(c0 + g * _L, _L)
            b16 = xv[s] + ov[s]
            k0 = g * (_L * _D)
            for d in range(_D):
                plsc.store_scatter(
                    idx_v,
                    [lanes16 + jnp.int32(k0 + d)],
                    b16 + jnp.int32(d * _V),
                )
            return 0

        lax.fori_loop(0, _CHUNK // _L, step, 0)

    build(0, idx_a)
    pltpu.async_copy(tab_hbm.at[idx_a], gbuf_a, sem_a)
    for c in range(_NCHUNK):
        cur = c % 2
        nxt = (c + 1) % 2
        if c + 1 < _NCHUNK:
            build(c + 1, idx_bufs[nxt])
        pltpu.make_async_copy(
            tab_hbm.at[idx_bufs[cur]], gbufs[cur], sems[cur]
        ).wait()
        if c + 1 < _NCHUNK:
            pltpu.async_copy(
                tab_hbm.at[idx_bufs[nxt]], gbufs[nxt], sems[nxt]
            )
        pltpu.sync_copy(
            gbufs[cur], out_hbm.at[pl.ds((base + c * _CHUNK) * _D, _CW)]
        )


@jax.jit
def kernel(x, tables):
    tab1d = tables.transpose(0, 1, 3, 2).reshape(_F * _E * _D * _V)
    x_flat = x.reshape(_ROWS)
    mesh = plsc.VectorSubcoreMesh(core_axis_name="c", subcore_axis_name="s")
    run = pl.kernel(
        _sc_body,
        mesh=mesh,
        out_type=jax.ShapeDtypeStruct((_ROWS * _D,), jnp.float32),
        scratch_types=[
            pltpu.VMEM((_RPW,), jnp.int32),
            pltpu.VMEM((_RPW,), jnp.int32),
            pltpu.VMEM((_CW,), jnp.int32),
            pltpu.VMEM((_CW,), jnp.int32),
            pltpu.VMEM((_CW,), jnp.float32),
            pltpu.VMEM((_CW,), jnp.float32),
            pltpu.SemaphoreType.DMA,
            pltpu.SemaphoreType.DMA,
        ],
        compiler_params=pltpu.CompilerParams(
            use_tc_tiling_on_sc=False, needs_layout_passes=False
        ),
    )
    out = run(tab1d, x_flat, jnp.asarray(_OFFSETS))
    return out.reshape(_B, _E, _F * _D)
